# initial kernel scaffold (unmeasured)
import jax
import jax.numpy as jnp
from jax import lax
from jax.experimental import pallas as pl
from jax.experimental.pallas import tpu as pltpu

N_DEV = 32


def kernel(x, w_mat):
    m_per, k = x.shape
    n = w_mat.shape[1]
    n_per = n // N_DEV
    m_total = m_per * N_DEV

    def body(x_ref, w_ref, out_ref, acc_ref, send_sems, recv_sems):
        me = lax.axis_index("i")

        y = jnp.dot(x_ref[...], w_ref[...], preferred_element_type=jnp.float32)
        acc_ref[...] = jnp.maximum(y, 0.0)

        out_ref[pl.ds(me * m_per, m_per), :] = acc_ref[:, pl.ds(me * n_per, n_per)]

        rdmas = []
        for off in range(1, N_DEV):
            tgt = (me + off) % N_DEV
            rdma = pltpu.make_async_remote_copy(
                src_ref=acc_ref.at[:, pl.ds(tgt * n_per, n_per)],
                dst_ref=out_ref.at[pl.ds(me * m_per, m_per), :],
                send_sem=send_sems.at[off],
                recv_sem=recv_sems.at[me],
                device_id=(tgt,),
                device_id_type=pl.DeviceIdType.MESH,
            )
            rdma.start()
            rdmas.append(rdma)

        for rdma in rdmas:
            rdma.wait_send()

        for off in range(1, N_DEV):
            src = (me - off) % N_DEV
            recv = pltpu.make_async_remote_copy(
                src_ref=acc_ref.at[:, pl.ds(src * n_per, n_per)],
                dst_ref=out_ref.at[pl.ds(src * m_per, m_per), :],
                send_sem=send_sems.at[off],
                recv_sem=recv_sems.at[src],
                device_id=(src,),
                device_id_type=pl.DeviceIdType.MESH,
            )
            recv.wait_recv()

    return pl.pallas_call(
        body,
        out_shape=jax.ShapeDtypeStruct((m_total, n_per), jnp.float32),
        in_specs=[
            pl.BlockSpec(memory_space=pltpu.VMEM),
            pl.BlockSpec(memory_space=pltpu.VMEM),
        ],
        out_specs=pl.BlockSpec(memory_space=pltpu.VMEM),
        scratch_shapes=[
            pltpu.VMEM((m_per, n), jnp.float32),
            pltpu.SemaphoreType.DMA((N_DEV,)),
            pltpu.SemaphoreType.DMA((N_DEV,)),
        ],
        compiler_params=pltpu.CompilerParams(collective_id=0),
    )(x, w_mat)


# baseline (device time: 54619 ns/iter reference)
import jax
import jax.numpy as jnp
from jax import lax
from jax.experimental import pallas as pl
from jax.experimental.pallas import tpu as pltpu

N_DEV = 32


def kernel(x, w_mat):
    m_per, k = x.shape
    n = w_mat.shape[1]
    n_per = n // N_DEV
    m_total = m_per * N_DEV

    def body(x_ref, w_ref, out_ref, acc_ref, send_sems, recv_sems):
        me = lax.axis_index("i")

        y = jnp.dot(x_ref[...], w_ref[...], preferred_element_type=jnp.float32)
        y = jnp.maximum(y, 0.0)
        for j in range(N_DEV):
            acc_ref[j] = y[:, j * n_per:(j + 1) * n_per]

        out_ref[pl.ds(me * m_per, m_per), :] = acc_ref[me]

        rdmas = []
        for off in range(1, N_DEV):
            tgt = (me + off) % N_DEV
            rdma = pltpu.make_async_remote_copy(
                src_ref=acc_ref.at[tgt],
                dst_ref=out_ref.at[pl.ds(me * m_per, m_per), :],
                send_sem=send_sems.at[off],
                recv_sem=recv_sems.at[me],
                device_id=(tgt,),
                device_id_type=pl.DeviceIdType.MESH,
            )
            rdma.start()
            rdmas.append(rdma)

        for rdma in rdmas:
            rdma.wait_send()

        for off in range(1, N_DEV):
            src = (me - off) % N_DEV
            recv = pltpu.make_async_remote_copy(
                src_ref=acc_ref.at[src],
                dst_ref=out_ref.at[pl.ds(src * m_per, m_per), :],
                send_sem=send_sems.at[off],
                recv_sem=recv_sems.at[src],
                device_id=(src,),
                device_id_type=pl.DeviceIdType.MESH,
            )
            recv.wait_recv()

    return pl.pallas_call(
        body,
        out_shape=jax.ShapeDtypeStruct((m_total, n_per), jnp.float32),
        in_specs=[
            pl.BlockSpec(memory_space=pltpu.VMEM),
            pl.BlockSpec(memory_space=pltpu.VMEM),
        ],
        out_specs=pl.BlockSpec(memory_space=pltpu.VMEM),
        scratch_shapes=[
            pltpu.VMEM((N_DEV, m_per, n_per), jnp.float32),
            pltpu.SemaphoreType.DMA((N_DEV,)),
            pltpu.SemaphoreType.DMA((N_DEV,)),
        ],
        compiler_params=pltpu.CompilerParams(
            vmem_limit_bytes=100 * 1024 * 1024,
        ),
    )(x, w_mat)
